# Initial kernel scaffold; baseline (speedup 1.0000x reference)
import jax
import jax.numpy as jnp
from jax import lax
from jax.experimental import pallas as pl
from jax.experimental.pallas import tpu as pltpu
from jax.experimental.pallas import tpu_sc as plsc

_NUM_CORES = 2
_NUM_SUBCORES = 16
_NW = _NUM_CORES * _NUM_SUBCORES
_LANES = 16


def _make_gather(B, D, R, K, b_per_w, chunk):
    n_chunks = b_per_w // chunk
    n_vec = b_per_w // _LANES
    mesh = plsc.VectorSubcoreMesh(
        core_axis_name="c",
        subcore_axis_name="s",
        num_cores=_NUM_CORES,
        num_subcores=_NUM_SUBCORES,
    )

    def body(table_hbm, idx_hbm, out_hbm, idx_v, gidx_v, rows_v, sem):
        wid = lax.axis_index("s") * _NUM_CORES + lax.axis_index("c")
        base = wid * b_per_w

        pltpu.sync_copy(idx_hbm.at[pl.ds(base, b_per_w)], idx_v)

        lanes = lax.iota(jnp.int32, _LANES)
        for j in range(n_vec):
            r = (base + j * _LANES) + lanes
            off = (r // K) * R
            gidx_v[pl.ds(j * _LANES, _LANES)] = (
                idx_v[pl.ds(j * _LANES, _LANES)] + off
            )

        for c in range(n_chunks):
            cb = c * chunk
            pltpu.async_copy(
                table_hbm.at[gidx_v.at[pl.ds(cb, chunk)]],
                rows_v,
                sem,
            ).wait()
            pltpu.sync_copy(rows_v, out_hbm.at[pl.ds(base + cb, chunk)])

    return pl.kernel(
        body,
        out_type=jax.ShapeDtypeStruct((B, D), jnp.float32),
        mesh=mesh,
        scratch_types=[
            pltpu.VMEM((b_per_w,), jnp.int32),
            pltpu.VMEM((b_per_w,), jnp.int32),
            pltpu.VMEM((chunk, D), jnp.float32),
            pltpu.SemaphoreType.DMA,
        ],
    )


def kernel(input_tensor, indices):
    N, R, D = input_tensor.shape
    K = indices.shape[1]
    B = N * K
    b_per_w = B // _NW
    assert B % _NW == 0 and b_per_w % _LANES == 0

    chunk = 8
    for cand in (128, 120, 112, 104, 96, 88, 80, 72, 64, 56, 48, 40, 32, 24, 16):
        if b_per_w % cand == 0:
            chunk = cand
            break

    table = input_tensor.reshape(N * R, D)
    idx_flat = indices.reshape(B).astype(jnp.int32)
    out = _make_gather(B, D, R, K, b_per_w, chunk)(table, idx_flat)
    return out.reshape(N, K, D)


# speedup vs baseline: 1.0584x; 1.0584x over previous
import jax
import jax.numpy as jnp
from jax import lax
from jax.experimental import pallas as pl
from jax.experimental.pallas import tpu as pltpu
from jax.experimental.pallas import tpu_sc as plsc

_NUM_CORES = 2
_NUM_SUBCORES = 16
_NW = _NUM_CORES * _NUM_SUBCORES
_LANES = 16


def _make_gather(B, D, R, K, b_per_w, chunk):
    n_chunks = b_per_w // chunk
    n_vec = b_per_w // _LANES
    mesh = plsc.VectorSubcoreMesh(
        core_axis_name="c",
        subcore_axis_name="s",
        num_cores=_NUM_CORES,
        num_subcores=_NUM_SUBCORES,
    )

    def body(table_hbm, idx_hbm, out_hbm, idx_v, gidx_v, rows_v, sem):
        wid = lax.axis_index("s") * _NUM_CORES + lax.axis_index("c")
        base = wid * b_per_w

        pltpu.sync_copy(idx_hbm.at[pl.ds(base, b_per_w)], idx_v)

        lanes = lax.iota(jnp.int32, _LANES)
        k_vec = jnp.full((_LANES,), K, jnp.int32)
        r_vec = jnp.full((_LANES,), R, jnp.int32)
        for j in range(n_vec):
            r = lanes + jnp.full((_LANES,), base + j * _LANES, jnp.int32)
            off = lax.div(r, k_vec) * r_vec
            gidx_v[pl.ds(j * _LANES, _LANES)] = (
                idx_v[pl.ds(j * _LANES, _LANES)] + off
            )

        for c in range(n_chunks):
            cb = c * chunk
            pltpu.async_copy(
                table_hbm.at[gidx_v.at[pl.ds(cb, chunk)]],
                rows_v,
                sem,
            ).wait()
            pltpu.sync_copy(rows_v, out_hbm.at[pl.ds(base + cb, chunk)])

    return pl.kernel(
        body,
        out_type=jax.ShapeDtypeStruct((B, D), jnp.float32),
        mesh=mesh,
        scratch_types=[
            pltpu.VMEM((b_per_w,), jnp.int32),
            pltpu.VMEM((b_per_w,), jnp.int32),
            pltpu.VMEM((chunk, D), jnp.float32),
            pltpu.SemaphoreType.DMA,
        ],
    )


def kernel(input_tensor, indices):
    N, R, D = input_tensor.shape
    K = indices.shape[1]
    B = N * K
    b_per_w = B // _NW
    assert B % _NW == 0 and b_per_w % _LANES == 0

    chunk = 8
    for cand in (128, 120, 112, 104, 96, 88, 80, 72, 64, 56, 48, 40, 32, 24, 16):
        if b_per_w % cand == 0:
            chunk = cand
            break

    table = input_tensor.reshape(N * R, D)
    idx_flat = indices.reshape(B).astype(jnp.int32)
    out = _make_gather(B, D, R, K, b_per_w, chunk)(table, idx_flat)
    return out.reshape(N, K, D)


# speedup vs baseline: 1.1959x; 1.1298x over previous
import jax
import jax.numpy as jnp
from jax import lax
from jax.experimental import pallas as pl
from jax.experimental.pallas import tpu as pltpu
from jax.experimental.pallas import tpu_sc as plsc

_NUM_CORES = 2
_NUM_SUBCORES = 16
_NW = _NUM_CORES * _NUM_SUBCORES
_LANES = 16


def _make_gather(B, D, R, K, b_per_w, chunk):
    n_chunks = b_per_w // chunk
    n_vec = b_per_w // _LANES
    mesh = plsc.VectorSubcoreMesh(
        core_axis_name="c",
        subcore_axis_name="s",
        num_cores=_NUM_CORES,
        num_subcores=_NUM_SUBCORES,
    )

    def body(table_hbm, idx_hbm, out_hbm, idx_v, gidx_v, rows_v, rows2_v,
             gsem0, gsem1, wsem0, wsem1):
        wid = lax.axis_index("s") * _NUM_CORES + lax.axis_index("c")
        base = wid * b_per_w

        pltpu.sync_copy(idx_hbm.at[pl.ds(base, b_per_w)], idx_v)

        lanes = lax.iota(jnp.int32, _LANES)
        k_vec = jnp.full((_LANES,), K, jnp.int32)
        r_vec = jnp.full((_LANES,), R, jnp.int32)
        for j in range(n_vec):
            r = lanes + jnp.full((_LANES,), base + j * _LANES, jnp.int32)
            off = lax.div(r, k_vec) * r_vec
            gidx_v[pl.ds(j * _LANES, _LANES)] = (
                idx_v[pl.ds(j * _LANES, _LANES)] + off
            )

        bufs = (rows_v, rows2_v)
        gsems = (gsem0, gsem1)
        wsems = (wsem0, wsem1)
        gathers = [None, None]
        writes = [None, None]

        def start_gather(c):
            s = c % 2
            gathers[s] = pltpu.async_copy(
                table_hbm.at[gidx_v.at[pl.ds(c * chunk, chunk)]],
                bufs[s],
                gsems[s],
            )

        start_gather(0)
        for c in range(n_chunks):
            s = c % 2
            if c + 1 < n_chunks:
                if writes[(c + 1) % 2] is not None:
                    writes[(c + 1) % 2].wait()
                start_gather(c + 1)
            gathers[s].wait()
            writes[s] = pltpu.async_copy(
                bufs[s],
                out_hbm.at[pl.ds(base + c * chunk, chunk)],
                wsems[s],
            )
        writes[(n_chunks - 1) % 2].wait()
        if writes[n_chunks % 2] is not None:
            writes[n_chunks % 2].wait()

    return pl.kernel(
        body,
        out_type=jax.ShapeDtypeStruct((B, D), jnp.float32),
        mesh=mesh,
        scratch_types=[
            pltpu.VMEM((b_per_w,), jnp.int32),
            pltpu.VMEM((b_per_w,), jnp.int32),
            pltpu.VMEM((chunk, D), jnp.float32),
            pltpu.VMEM((chunk, D), jnp.float32),
            pltpu.SemaphoreType.DMA,
            pltpu.SemaphoreType.DMA,
            pltpu.SemaphoreType.DMA,
            pltpu.SemaphoreType.DMA,
        ],
    )


def kernel(input_tensor, indices):
    N, R, D = input_tensor.shape
    K = indices.shape[1]
    B = N * K
    b_per_w = B // _NW
    assert B % _NW == 0 and b_per_w % _LANES == 0

    chunk = 8
    for cand in (128, 120, 112, 104, 96, 88, 80, 72, 64, 56, 48, 40, 32, 24, 16):
        if b_per_w % cand == 0:
            chunk = cand
            break

    table = input_tensor.reshape(N * R, D)
    idx_flat = indices.reshape(B).astype(jnp.int32)
    out = _make_gather(B, D, R, K, b_per_w, chunk)(table, idx_flat)
    return out.reshape(N, K, D)
